# baseline (speedup 1.0000x reference)
import functools

import jax
import jax.numpy as jnp
from jax import lax
from jax.experimental import pallas as pl
from jax.experimental.pallas import tpu as pltpu
from jax.experimental.pallas import tpu_sc as plsc

H, W = 1024, 1024
NC, NS, L = 2, 16, 16
NW = NC * NS
BAND = (H * W) // NW
NPAD = 2 ** 21
P = 8192
CB = 1024
S = 8192


def _proj_body(p0_ref, p1_ref, p2_ref, flat_ref, depth_ref):
    p0 = p0_ref[...]
    p1 = p1_ref[...]
    p2 = p2_ref[...]
    d = p2 + 1e-8
    px = jnp.floor(p0 / d).astype(jnp.int32)
    py = jnp.floor(p1 / d).astype(jnp.int32)
    valid = (px >= 0) & (px < W) & (py >= 0) & (py < H) & (p2 > 0)
    flat_ref[...] = jnp.where(valid, py * W + px, H * W)
    depth_ref[...] = jnp.where(valid, p2, jnp.inf)


def _project(p0, p1, p2):
    n = p0.shape[0]
    bn = 131072
    return pl.pallas_call(
        _proj_body,
        grid=(n // bn,),
        in_specs=[
            pl.BlockSpec((bn,), lambda i: (i,)),
            pl.BlockSpec((bn,), lambda i: (i,)),
            pl.BlockSpec((bn,), lambda i: (i,)),
        ],
        out_specs=[
            pl.BlockSpec((bn,), lambda i: (i,)),
            pl.BlockSpec((bn,), lambda i: (i,)),
        ],
        out_shape=[
            jax.ShapeDtypeStruct((n,), jnp.int32),
            jax.ShapeDtypeStruct((n,), jnp.float32),
        ],
    )(p0, p1, p2)


def _zbuffer_call(n):
    mesh = plsc.VectorSubcoreMesh(
        core_axis_name="c", subcore_axis_name="s", num_cores=NC, num_subcores=NS
    )

    @functools.partial(
        pl.kernel,
        out_type=jax.ShapeDtypeStruct((3, H * W), jnp.float32),
        mesh=mesh,
        scratch_types=[
            pltpu.VMEM((P,), jnp.int32),
            pltpu.VMEM((P,), jnp.float32),
            pltpu.VMEM((BAND,), jnp.float32),
            pltpu.VMEM((BAND // CB, CB), jnp.int32),
            pltpu.VMEM((3, CB), jnp.float32),
            pltpu.VMEM((S,), jnp.int32),
            pltpu.VMEM((S,), jnp.float32),
            pltpu.VMEM((S,), jnp.int32),
            pltpu.SemaphoreType.DMA,
        ],
        compiler_params=pltpu.CompilerParams(
            needs_layout_passes=False, use_tc_tiling_on_sc=False
        ),
    )
    def zkern(flat_hbm, depth_hbm, colp_hbm, out_hbm, fbuf, dbuf, zbuf, widx,
              cbuf, sli, sdp, sgi, sem):
        c = lax.axis_index("c")
        s = lax.axis_index("s")
        wid = s * NC + c
        lo = wid * BAND
        hi = lo + BAND
        iot = lax.iota(jnp.int32, L)

        def init_i(i, carry):
            zbuf[pl.ds(i * L, L)] = jnp.full((L,), jnp.inf, jnp.float32)
            widx[(i * L) // CB, pl.ds((i * L) % CB, L)] = jnp.full((L,), n, jnp.int32)
            return carry

        lax.fori_loop(0, BAND // L, init_i, 0)

        def drain(cnt):
            nv = (cnt + (L - 1)) // L

            def dbody(j, carry):
                mm = iot < (cnt - j * L)
                li = sli[pl.ds(j * L, L)] & (BAND - 1)
                dp = sdp[pl.ds(j * L, L)]
                gi = sgi[pl.ds(j * L, L)]
                cur = plsc.load_gather(zbuf, [li])
                win0 = (dp < cur) & mm

                def wcond(cr):
                    return plsc.all_reduce_population_count(cr[0])[0] > 0

                def wbody(cr):
                    wn, _ = cr
                    plsc.store_scatter(zbuf, [li], dp, mask=wn)
                    cur2 = plsc.load_gather(zbuf, [li])
                    return (dp < cur2) & mm, cur2

                _, curf = lax.while_loop(wcond, wbody, (win0, cur))
                fw = (dp == curf) & mm
                plsc.store_scatter(widx, [li // CB, li & (CB - 1)], gi, mask=fw)
                return carry

            lax.fori_loop(0, nv, dbody, 0)

        def chunk_body(ci, carry):
            base = ci * P
            pltpu.sync_copy(flat_hbm.at[pl.ds(base, P)], fbuf)
            pltpu.sync_copy(depth_hbm.at[pl.ds(base, P)], dbuf)

            def vec_body(j, cntv):
                fl = fbuf[pl.ds(j * L, L)]
                dp = dbuf[pl.ds(j * L, L)]
                li = fl - lo
                m = li.astype(jnp.uint32) < jnp.uint32(BAND)
                gi = (base + j * L) + iot
                mi = m.astype(jnp.int32)
                pos = cntv + plsc.cumsum(mi) - mi
                plsc.store_scatter(sli, [pos], li, mask=m)
                plsc.store_scatter(sdp, [pos], dp, mask=m)
                plsc.store_scatter(sgi, [pos], gi, mask=m)
                return cntv + plsc.all_reduce_population_count(m)

            cntv = plsc.parallel_loop(
                0, P // L, 1, unroll=8, carry=jnp.zeros((L,), jnp.int32)
            )(lambda j, cv: vec_body(j, cv))
            drain(cntv[0])
            return carry

        lax.fori_loop(0, NPAD // P, chunk_body, 0)

        def col_body(k, carry):
            d0 = pltpu.async_copy(colp_hbm.at[0].at[widx.at[k]], cbuf.at[0], sem)
            d1 = pltpu.async_copy(colp_hbm.at[1].at[widx.at[k]], cbuf.at[1], sem)
            d2 = pltpu.async_copy(colp_hbm.at[2].at[widx.at[k]], cbuf.at[2], sem)
            d0.wait()
            d1.wait()
            d2.wait()
            pltpu.sync_copy(cbuf.at[0], out_hbm.at[0].at[pl.ds(lo + k * CB, CB)])
            pltpu.sync_copy(cbuf.at[1], out_hbm.at[1].at[pl.ds(lo + k * CB, CB)])
            pltpu.sync_copy(cbuf.at[2], out_hbm.at[2].at[pl.ds(lo + k * CB, CB)])
            return carry

        lax.fori_loop(0, BAND // CB, col_body, 0)

    return zkern


def kernel(points, colors, extrinsic, intrinsic):
    n = points.shape[0]
    positions = jnp.concatenate([points, jnp.ones((n, 1), points.dtype)], axis=1)
    inv_ext = jnp.linalg.inv(extrinsic)
    proj = (intrinsic @ (inv_ext[0:3] @ positions.T)).T
    pad = NPAD - n
    projp = jnp.pad(proj, ((0, pad), (0, 0)))
    flat, depth = _project(projp[:, 0], projp[:, 1], projp[:, 2])
    colp = jnp.concatenate([colors, jnp.ones((1, 3), jnp.float32)], axis=0).T
    img3 = _zbuffer_call(n)(flat, depth, colp)
    return img3.T.reshape(H, W, 3)


# speedup vs baseline: 1.0057x; 1.0057x over previous
import functools

import jax
import jax.numpy as jnp
from jax import lax
from jax.experimental import pallas as pl
from jax.experimental.pallas import tpu as pltpu
from jax.experimental.pallas import tpu_sc as plsc

H, W = 1024, 1024
NC, NS, L = 2, 16, 16
NW = NC * NS
BAND = (H * W) // NW
NPAD = 2 ** 21
P = 8192
CB = 1024
S = 8192


def _proj_body(p0_ref, p1_ref, p2_ref, flat_ref, depth_ref):
    p0 = p0_ref[...]
    p1 = p1_ref[...]
    p2 = p2_ref[...]
    d = p2 + 1e-8
    px = jnp.floor(p0 / d).astype(jnp.int32)
    py = jnp.floor(p1 / d).astype(jnp.int32)
    valid = (px >= 0) & (px < W) & (py >= 0) & (py < H) & (p2 > 0)
    flat_ref[...] = jnp.where(valid, py * W + px, H * W)
    depth_ref[...] = jnp.where(valid, p2, jnp.inf)


def _project(p0, p1, p2):
    n = p0.shape[0]
    bn = 131072
    return pl.pallas_call(
        _proj_body,
        grid=(n // bn,),
        in_specs=[
            pl.BlockSpec((bn,), lambda i: (i,)),
            pl.BlockSpec((bn,), lambda i: (i,)),
            pl.BlockSpec((bn,), lambda i: (i,)),
        ],
        out_specs=[
            pl.BlockSpec((bn,), lambda i: (i,)),
            pl.BlockSpec((bn,), lambda i: (i,)),
        ],
        out_shape=[
            jax.ShapeDtypeStruct((n,), jnp.int32),
            jax.ShapeDtypeStruct((n,), jnp.float32),
        ],
    )(p0, p1, p2)


def _zbuffer_call(n):
    mesh = plsc.VectorSubcoreMesh(
        core_axis_name="c", subcore_axis_name="s", num_cores=NC, num_subcores=NS
    )

    @functools.partial(
        pl.kernel,
        out_type=jax.ShapeDtypeStruct((3, H * W), jnp.float32),
        mesh=mesh,
        scratch_types=[
            pltpu.VMEM((P,), jnp.int32),
            pltpu.VMEM((P,), jnp.float32),
            pltpu.VMEM((BAND,), jnp.float32),
            pltpu.VMEM((BAND // CB, CB), jnp.int32),
            pltpu.VMEM((3, CB), jnp.float32),
            pltpu.VMEM((S,), jnp.int32),
            pltpu.VMEM((S,), jnp.float32),
            pltpu.VMEM((S,), jnp.int32),
            pltpu.SemaphoreType.DMA,
        ],
        compiler_params=pltpu.CompilerParams(
            needs_layout_passes=False, use_tc_tiling_on_sc=False
        ),
    )
    def zkern(flat_hbm, depth_hbm, colp_hbm, out_hbm, fbuf, dbuf, zbuf, widx,
              cbuf, sli, sdp, sgi, sem):
        c = lax.axis_index("c")
        s = lax.axis_index("s")
        wid = s * NC + c
        lo = wid * BAND
        hi = lo + BAND
        iot = lax.iota(jnp.int32, L)

        def init_i(i, carry):
            zbuf[pl.ds(i * L, L)] = jnp.full((L,), jnp.inf, jnp.float32)
            widx[(i * L) // CB, pl.ds((i * L) % CB, L)] = jnp.full((L,), n, jnp.int32)
            return carry

        lax.fori_loop(0, BAND // L, init_i, 0)

        def drain(cnt):
            nv = (cnt + (L - 1)) // L

            def dbody(j, carry):
                mm = iot < (cnt - j * L)
                li = sli[pl.ds(j * L, L)] & (BAND - 1)
                dp = sdp[pl.ds(j * L, L)]
                gi = sgi[pl.ds(j * L, L)]
                cur = plsc.load_gather(zbuf, [li])
                win0 = (dp < cur) & mm

                def wcond(cr):
                    return plsc.all_reduce_population_count(cr[0])[0] > 0

                def wbody(cr):
                    wn, _ = cr
                    plsc.store_scatter(zbuf, [li], dp, mask=wn)
                    cur2 = plsc.load_gather(zbuf, [li])
                    return (dp < cur2) & mm, cur2

                _, curf = lax.while_loop(wcond, wbody, (win0, cur))
                fw = (dp == curf) & mm
                plsc.store_scatter(widx, [li // CB, li & (CB - 1)], gi, mask=fw)
                return carry

            lax.fori_loop(0, nv, dbody, 0)

        def chunk_body(ci, carry):
            base = ci * P
            pltpu.sync_copy(flat_hbm.at[pl.ds(base, P)], fbuf)
            pltpu.sync_copy(depth_hbm.at[pl.ds(base, P)], dbuf)

            def vec_body(j, cnt2):
                fl = fbuf[pl.ds(j * L, L)]
                dp = dbuf[pl.ds(j * L, L)]
                li = fl - lo
                m = li.astype(jnp.uint32) < jnp.uint32(BAND)
                gi = (base + j * L) + iot
                plsc.store_compressed(sli.at[pl.ds(cnt2, L)], li, mask=m)
                plsc.store_compressed(sdp.at[pl.ds(cnt2, L)], dp, mask=m)
                plsc.store_compressed(sgi.at[pl.ds(cnt2, L)], gi, mask=m)
                return cnt2 + plsc.all_reduce_population_count(m)[0]

            cnt = plsc.parallel_loop(0, P // L, 1, unroll=8, carry=jnp.int32(0))(
                lambda j, cnt2: vec_body(j, cnt2)
            )
            drain(cnt)
            return carry

        lax.fori_loop(0, NPAD // P, chunk_body, 0)

        def col_body(k, carry):
            d0 = pltpu.async_copy(colp_hbm.at[0].at[widx.at[k]], cbuf.at[0], sem)
            d1 = pltpu.async_copy(colp_hbm.at[1].at[widx.at[k]], cbuf.at[1], sem)
            d2 = pltpu.async_copy(colp_hbm.at[2].at[widx.at[k]], cbuf.at[2], sem)
            d0.wait()
            d1.wait()
            d2.wait()
            pltpu.sync_copy(cbuf.at[0], out_hbm.at[0].at[pl.ds(lo + k * CB, CB)])
            pltpu.sync_copy(cbuf.at[1], out_hbm.at[1].at[pl.ds(lo + k * CB, CB)])
            pltpu.sync_copy(cbuf.at[2], out_hbm.at[2].at[pl.ds(lo + k * CB, CB)])
            return carry

        lax.fori_loop(0, BAND // CB, col_body, 0)

    return zkern


def kernel(points, colors, extrinsic, intrinsic):
    n = points.shape[0]
    positions = jnp.concatenate([points, jnp.ones((n, 1), points.dtype)], axis=1)
    inv_ext = jnp.linalg.inv(extrinsic)
    proj = (intrinsic @ (inv_ext[0:3] @ positions.T)).T
    pad = NPAD - n
    projp = jnp.pad(proj, ((0, pad), (0, 0)))
    flat, depth = _project(projp[:, 0], projp[:, 1], projp[:, 2])
    colp = jnp.concatenate([colors, jnp.ones((1, 3), jnp.float32)], axis=0).T
    img3 = _zbuffer_call(n)(flat, depth, colp)
    return img3.T.reshape(H, W, 3)
